# baseline (speedup 1.0000x reference)
import functools

import jax
import jax.numpy as jnp
from jax.experimental import pallas as pl
from jax.experimental.pallas import tpu as pltpu


def _pick_block(m, candidates):
    for c in candidates:
        if m % c == 0:
            return c
    return m


def _leaky(v):
    return jnp.where(v >= 0.0, v, 0.01 * v)


def _support0_kernel(act_ref, x_ref, w_ref, o_ref):
    s = jnp.dot(x_ref[...], w_ref[...], preferred_element_type=jnp.float32)
    s = jnp.where(act_ref[0, 0] != 0, _leaky(s), s)
    o_ref[...] = s.astype(o_ref.dtype)


def _prop_w_kernel(act_ref, adj_ref, s_ref, w_ref, o_ref, *, leaky):
    z = jnp.dot(adj_ref[...], s_ref[...], preferred_element_type=jnp.float32)
    z = jnp.dot(z, w_ref[...], preferred_element_type=jnp.float32)
    if leaky:
        z = jnp.where(act_ref[0, 0] != 0, _leaky(z), z)
    o_ref[...] = z.astype(o_ref.dtype)


def _prop_id_kernel(adj_ref, s_ref, o_ref):
    o_ref[...] = jnp.dot(
        adj_ref[...], s_ref[...], preferred_element_type=jnp.float32
    )


def _gram_kernel(a_ref, bt_ref, o_ref):
    a = a_ref[...].astype(jnp.bfloat16)
    bt = bt_ref[...].astype(jnp.bfloat16)
    g = jnp.dot(a, bt, preferred_element_type=jnp.float32)
    o_ref[...] = jax.nn.sigmoid(g)


def _support0(x, w1t, act):
    m, _ = x.shape
    n = w1t.shape[1]
    return pl.pallas_call(
        _support0_kernel,
        in_specs=[
            pl.BlockSpec(memory_space=pltpu.SMEM),
            pl.BlockSpec((m, x.shape[1]), lambda: (0, 0)),
            pl.BlockSpec((x.shape[1], n), lambda: (0, 0)),
        ],
        out_specs=pl.BlockSpec((m, n), lambda: (0, 0)),
        out_shape=jax.ShapeDtypeStruct((m, n), jnp.bfloat16),
    )(act, x, w1t)


def _propagate(adj16, s, wt, act, *, leaky, out_dtype):
    m, k = adj16.shape
    n = s.shape[1]
    bm = _pick_block(m, (400, 200, 80, 16, 8))
    grid = (m // bm,)
    if wt is not None:
        n2 = wt.shape[1]
        return pl.pallas_call(
            functools.partial(_prop_w_kernel, leaky=leaky),
            grid=grid,
            in_specs=[
                pl.BlockSpec(memory_space=pltpu.SMEM),
                pl.BlockSpec((bm, k), lambda i: (i, 0)),
                pl.BlockSpec((k, n), lambda i: (0, 0)),
                pl.BlockSpec((n, n2), lambda i: (0, 0)),
            ],
            out_specs=pl.BlockSpec((bm, n2), lambda i: (i, 0)),
            out_shape=jax.ShapeDtypeStruct((m, n2), out_dtype),
        )(act, adj16, s, wt)
    return pl.pallas_call(
        _prop_id_kernel,
        grid=grid,
        in_specs=[
            pl.BlockSpec((bm, k), lambda i: (i, 0)),
            pl.BlockSpec((k, n), lambda i: (0, 0)),
        ],
        out_specs=pl.BlockSpec((bm, n), lambda i: (i, 0)),
        out_shape=jax.ShapeDtypeStruct((m, n), out_dtype),
    )(adj16, s)


def _gram_sigmoid(z, zt):
    m = z.shape[0]
    k = z.shape[1]
    bm = _pick_block(m, (400, 200, 80, 16, 8))
    return pl.pallas_call(
        _gram_kernel,
        grid=(m // bm,),
        in_specs=[
            pl.BlockSpec((bm, k), lambda i: (i, 0)),
            pl.BlockSpec((k, m), lambda i: (0, 0)),
        ],
        out_specs=pl.BlockSpec((bm, m), lambda i: (i, 0)),
        out_shape=jax.ShapeDtypeStruct((m, m), jnp.float32),
    )(z, zt)


def kernel(x, adj, active, W1, W2, W3):
    act = jnp.asarray(active, jnp.int32).reshape(1, 1)
    adj16 = adj.astype(jnp.bfloat16)
    s1 = _support0(x, W1.T, act)
    s2 = _propagate(adj16, s1, W2.T, act, leaky=True, out_dtype=jnp.bfloat16)
    s3 = _propagate(adj16, s2, W3.T, act, leaky=False, out_dtype=jnp.bfloat16)
    z3 = _propagate(adj16, s3, None, None, leaky=False, out_dtype=jnp.float32)
    adj_hat = _gram_sigmoid(z3, z3.T)
    return (z3, adj_hat)


# speedup vs baseline: 1.1452x; 1.1452x over previous
import functools

import jax
import jax.numpy as jnp
from jax.experimental import pallas as pl
from jax.experimental.pallas import tpu as pltpu


def _pick_block(m, candidates):
    for c in candidates:
        if m % c == 0:
            return c
    return m


def _leaky(v):
    return jnp.where(v >= 0.0, v, 0.01 * v)


def _support0_kernel(act_ref, x_ref, w_ref, o_ref):
    s = jnp.dot(x_ref[...], w_ref[...], preferred_element_type=jnp.float32)
    s = jnp.where(act_ref[0, 0] != 0, _leaky(s), s)
    o_ref[...] = s.astype(o_ref.dtype)


def _prop_w_kernel(act_ref, adj_ref, s_ref, w_ref, o_ref, *, leaky):
    z = jnp.dot(
        adj_ref[...].astype(jnp.bfloat16), s_ref[...],
        preferred_element_type=jnp.float32,
    )
    z = jnp.dot(z, w_ref[...], preferred_element_type=jnp.float32)
    if leaky:
        z = jnp.where(act_ref[0, 0] != 0, _leaky(z), z)
    o_ref[...] = z.astype(o_ref.dtype)


def _prop_w_cast_kernel(act_ref, adj_ref, s_ref, w_ref, o_ref, adj_c_ref, *, leaky):
    adj_blk = adj_ref[...]
    adj_c_ref[...] = adj_blk.astype(adj_c_ref.dtype)
    z = jnp.dot(
        adj_blk.astype(jnp.bfloat16), s_ref[...],
        preferred_element_type=jnp.float32,
    )
    z = jnp.dot(z, w_ref[...], preferred_element_type=jnp.float32)
    if leaky:
        z = jnp.where(act_ref[0, 0] != 0, _leaky(z), z)
    o_ref[...] = z.astype(o_ref.dtype)


def _prop_id_kernel(adj_ref, s_ref, o_ref):
    o_ref[...] = jnp.dot(
        adj_ref[...].astype(jnp.bfloat16), s_ref[...],
        preferred_element_type=jnp.float32,
    )


def _gram_kernel(a_ref, bt_ref, o_ref):
    a = a_ref[...].astype(jnp.bfloat16)
    bt = bt_ref[...].astype(jnp.bfloat16)
    g = jnp.dot(a, bt, preferred_element_type=jnp.float32)
    o_ref[...] = jax.nn.sigmoid(g)


def _support0(x, w1t, act):
    m, _ = x.shape
    n = w1t.shape[1]
    return pl.pallas_call(
        _support0_kernel,
        in_specs=[
            pl.BlockSpec(memory_space=pltpu.SMEM),
            pl.BlockSpec((m, x.shape[1]), lambda: (0, 0)),
            pl.BlockSpec((x.shape[1], n), lambda: (0, 0)),
        ],
        out_specs=pl.BlockSpec((m, n), lambda: (0, 0)),
        out_shape=jax.ShapeDtypeStruct((m, n), jnp.bfloat16),
    )(act, x, w1t)


def _propagate_cast(adj, s, wt, act, *, leaky, out_dtype, compact_dtype):
    m, k = adj.shape
    n = s.shape[1]
    n2 = wt.shape[1]
    bm = _pick_block(m, (200, 80, 16, 8))
    grid = (m // bm,)
    return pl.pallas_call(
        functools.partial(_prop_w_cast_kernel, leaky=leaky),
        grid=grid,
        in_specs=[
            pl.BlockSpec(memory_space=pltpu.SMEM),
            pl.BlockSpec((bm, k), lambda i: (i, 0)),
            pl.BlockSpec((k, n), lambda i: (0, 0)),
            pl.BlockSpec((n, n2), lambda i: (0, 0)),
        ],
        out_specs=[
            pl.BlockSpec((bm, n2), lambda i: (i, 0)),
            pl.BlockSpec((bm, k), lambda i: (i, 0)),
        ],
        out_shape=[
            jax.ShapeDtypeStruct((m, n2), out_dtype),
            jax.ShapeDtypeStruct((m, k), compact_dtype),
        ],
    )(act, adj, s, wt)


def _propagate(adj16, s, wt, act, *, leaky, out_dtype):
    m, k = adj16.shape
    n = s.shape[1]
    bm = _pick_block(m, (400, 200, 80, 16, 8))
    grid = (m // bm,)
    if wt is not None:
        n2 = wt.shape[1]
        return pl.pallas_call(
            functools.partial(_prop_w_kernel, leaky=leaky),
            grid=grid,
            in_specs=[
                pl.BlockSpec(memory_space=pltpu.SMEM),
                pl.BlockSpec((bm, k), lambda i: (i, 0)),
                pl.BlockSpec((k, n), lambda i: (0, 0)),
                pl.BlockSpec((n, n2), lambda i: (0, 0)),
            ],
            out_specs=pl.BlockSpec((bm, n2), lambda i: (i, 0)),
            out_shape=jax.ShapeDtypeStruct((m, n2), out_dtype),
        )(act, adj16, s, wt)
    return pl.pallas_call(
        _prop_id_kernel,
        grid=grid,
        in_specs=[
            pl.BlockSpec((bm, k), lambda i: (i, 0)),
            pl.BlockSpec((k, n), lambda i: (0, 0)),
        ],
        out_specs=pl.BlockSpec((bm, n), lambda i: (i, 0)),
        out_shape=jax.ShapeDtypeStruct((m, n), out_dtype),
    )(adj16, s)


def _gram_sigmoid(z, zt):
    m = z.shape[0]
    k = z.shape[1]
    bm = _pick_block(m, (400, 200, 80, 16, 8))
    return pl.pallas_call(
        _gram_kernel,
        grid=(m // bm,),
        in_specs=[
            pl.BlockSpec((bm, k), lambda i: (i, 0)),
            pl.BlockSpec((k, m), lambda i: (0, 0)),
        ],
        out_specs=pl.BlockSpec((bm, m), lambda i: (i, 0)),
        out_shape=jax.ShapeDtypeStruct((m, m), jnp.float32),
    )(z, zt)


def kernel(x, adj, active, W1, W2, W3):
    act = jnp.asarray(active, jnp.int32).reshape(1, 1)
    s1 = _support0(x, W1.T, act)
    s2, adj_c = _propagate_cast(
        adj, s1, W2.T, act, leaky=True, out_dtype=jnp.bfloat16,
        compact_dtype=jnp.bfloat16,
    )
    s3 = _propagate(adj_c, s2, W3.T, act, leaky=False, out_dtype=jnp.bfloat16)
    z3 = _propagate(adj_c, s3, None, None, leaky=False, out_dtype=jnp.float32)
    adj_hat = _gram_sigmoid(z3, z3.T)
    return (z3, adj_hat)


# speedup vs baseline: 1.3029x; 1.1376x over previous
import functools

import jax
import jax.numpy as jnp
from jax.experimental import pallas as pl
from jax.experimental.pallas import tpu as pltpu


def _pick_block(m, candidates):
    for c in candidates:
        if m % c == 0:
            return c
    return m


def _leaky(v):
    return jnp.where(v >= 0.0, v, 0.01 * v)


def _support0_kernel(act_ref, x_ref, w_ref, o_ref):
    s = jnp.dot(x_ref[...], w_ref[...], preferred_element_type=jnp.float32)
    s = jnp.where(act_ref[0, 0] != 0, _leaky(s), s)
    o_ref[...] = s.astype(o_ref.dtype)


def _prop_w_kernel(act_ref, adj_ref, s_ref, w_ref, o_ref, *, leaky):
    z = jnp.dot(
        adj_ref[...].astype(jnp.bfloat16), s_ref[...],
        preferred_element_type=jnp.float32,
    )
    z = jnp.dot(z, w_ref[...], preferred_element_type=jnp.float32)
    if leaky:
        z = jnp.where(act_ref[0, 0] != 0, _leaky(z), z)
    o_ref[...] = z.astype(o_ref.dtype)


def _prop_w_cast_kernel(act_ref, adj_ref, s_ref, w_ref, o_ref, adj_c_ref, *, leaky):
    adj_blk = adj_ref[...]
    adj_c_ref[...] = adj_blk.astype(adj_c_ref.dtype)
    z = jnp.dot(
        adj_blk.astype(jnp.bfloat16), s_ref[...],
        preferred_element_type=jnp.float32,
    )
    z = jnp.dot(z, w_ref[...], preferred_element_type=jnp.float32)
    if leaky:
        z = jnp.where(act_ref[0, 0] != 0, _leaky(z), z)
    o_ref[...] = z.astype(o_ref.dtype)


def _prop_id_kernel(adj_ref, s_ref, o_ref):
    o_ref[...] = jnp.dot(
        adj_ref[...].astype(jnp.bfloat16), s_ref[...],
        preferred_element_type=jnp.float32,
    )


def _gram_kernel(a_ref, bt_ref, o_ref):
    a = a_ref[...].astype(jnp.bfloat16)
    bt = bt_ref[...].astype(jnp.bfloat16)
    g = jnp.dot(a, bt, preferred_element_type=jnp.float32)
    o_ref[...] = jax.nn.sigmoid(g)


def _support0(x, w1t, act):
    m, _ = x.shape
    n = w1t.shape[1]
    return pl.pallas_call(
        _support0_kernel,
        in_specs=[
            pl.BlockSpec(memory_space=pltpu.SMEM),
            pl.BlockSpec((m, x.shape[1]), lambda: (0, 0)),
            pl.BlockSpec((x.shape[1], n), lambda: (0, 0)),
        ],
        out_specs=pl.BlockSpec((m, n), lambda: (0, 0)),
        out_shape=jax.ShapeDtypeStruct((m, n), jnp.bfloat16),
    )(act, x, w1t)


def _propagate_cast(adj, s, wt, act, *, leaky, out_dtype, compact_dtype):
    m, k = adj.shape
    n = s.shape[1]
    n2 = wt.shape[1]
    bm = _pick_block(m, (200, 80, 16, 8))
    grid = (m // bm,)
    return pl.pallas_call(
        functools.partial(_prop_w_cast_kernel, leaky=leaky),
        grid=grid,
        in_specs=[
            pl.BlockSpec(memory_space=pltpu.SMEM),
            pl.BlockSpec((bm, k), lambda i: (i, 0)),
            pl.BlockSpec((k, n), lambda i: (0, 0)),
            pl.BlockSpec((n, n2), lambda i: (0, 0)),
        ],
        out_specs=[
            pl.BlockSpec((bm, n2), lambda i: (i, 0)),
            pl.BlockSpec((bm, k), lambda i: (i, 0)),
        ],
        out_shape=[
            jax.ShapeDtypeStruct((m, n2), out_dtype),
            jax.ShapeDtypeStruct((m, k), compact_dtype),
        ],
    )(act, adj, s, wt)


def _propagate(adj16, s, wt, act, *, leaky, out_dtype):
    m, k = adj16.shape
    n = s.shape[1]
    bm = _pick_block(m, (400, 200, 80, 16, 8))
    grid = (m // bm,)
    if wt is not None:
        n2 = wt.shape[1]
        return pl.pallas_call(
            functools.partial(_prop_w_kernel, leaky=leaky),
            grid=grid,
            in_specs=[
                pl.BlockSpec(memory_space=pltpu.SMEM),
                pl.BlockSpec((bm, k), lambda i: (i, 0)),
                pl.BlockSpec((k, n), lambda i: (0, 0)),
                pl.BlockSpec((n, n2), lambda i: (0, 0)),
            ],
            out_specs=pl.BlockSpec((bm, n2), lambda i: (i, 0)),
            out_shape=jax.ShapeDtypeStruct((m, n2), out_dtype),
        )(act, adj16, s, wt)
    return pl.pallas_call(
        _prop_id_kernel,
        grid=grid,
        in_specs=[
            pl.BlockSpec((bm, k), lambda i: (i, 0)),
            pl.BlockSpec((k, n), lambda i: (0, 0)),
        ],
        out_specs=pl.BlockSpec((bm, n), lambda i: (i, 0)),
        out_shape=jax.ShapeDtypeStruct((m, n), out_dtype),
    )(adj16, s)


def _gram_sigmoid(z, zt):
    m = z.shape[0]
    k = z.shape[1]
    bm = _pick_block(m, (400, 200, 80, 16, 8))
    return pl.pallas_call(
        _gram_kernel,
        grid=(m // bm,),
        in_specs=[
            pl.BlockSpec((bm, k), lambda i: (i, 0)),
            pl.BlockSpec((k, m), lambda i: (0, 0)),
        ],
        out_specs=pl.BlockSpec((bm, m), lambda i: (i, 0)),
        out_shape=jax.ShapeDtypeStruct((m, m), jnp.float32),
    )(z, zt)


def kernel(x, adj, active, W1, W2, W3):
    act = jnp.asarray(active, jnp.int32).reshape(1, 1)
    s1 = _support0(x, W1.T, act)
    s2, adj_c = _propagate_cast(
        adj, s1, W2.T, act, leaky=True, out_dtype=jnp.bfloat16,
        compact_dtype=jnp.float8_e4m3fn,
    )
    s3 = _propagate(adj_c, s2, W3.T, act, leaky=False, out_dtype=jnp.bfloat16)
    z3 = _propagate(adj_c, s3, None, None, leaky=False, out_dtype=jnp.float32)
    adj_hat = _gram_sigmoid(z3, z3.T)
    return (z3, adj_hat)


# speedup vs baseline: 1.3376x; 1.0267x over previous
import functools

import jax
import jax.numpy as jnp
from jax.experimental import pallas as pl
from jax.experimental.pallas import tpu as pltpu


def _pick_block(m, candidates):
    for c in candidates:
        if m % c == 0:
            return c
    return m


def _leaky(v):
    return jnp.where(v >= 0.0, v, 0.01 * v)


def _support0_kernel(act_ref, x_ref, w_ref, o_ref):
    s = jnp.dot(x_ref[...], w_ref[...], preferred_element_type=jnp.float32)
    s = jnp.where(act_ref[0, 0] != 0, _leaky(s), s)
    o_ref[...] = s.astype(o_ref.dtype)


def _prop_w_kernel(act_ref, scale_ref, adj_ref, s_ref, w_ref, o_ref, *, leaky):
    z = jax.lax.dot_general(
        adj_ref[...], s_ref[...], (((1,), (0,)), ((), ())),
        preferred_element_type=jnp.float32,
    )
    z = z * scale_ref[...]
    n = z.shape[1] // 2
    z = z[:, :n] + z[:, n:]
    z = jnp.dot(z, w_ref[...], preferred_element_type=jnp.float32)
    if leaky:
        z = jnp.where(act_ref[0, 0] != 0, _leaky(z), z)
    o_ref[...] = z.astype(o_ref.dtype)


def _prop_w_cast_kernel(act_ref, adj_ref, s_ref, w_ref, o_ref, adj_c_ref, *, leaky):
    adj_blk = adj_ref[...]
    adj_c_ref[...] = adj_blk.astype(adj_c_ref.dtype)
    z = jnp.dot(
        adj_blk.astype(jnp.bfloat16), s_ref[...],
        preferred_element_type=jnp.float32,
    )
    z = jnp.dot(z, w_ref[...], preferred_element_type=jnp.float32)
    if leaky:
        z = jnp.where(act_ref[0, 0] != 0, _leaky(z), z)
    o_ref[...] = z.astype(o_ref.dtype)


def _prop_id_kernel(scale_ref, adj_ref, s_ref, o_ref):
    z = jax.lax.dot_general(
        adj_ref[...], s_ref[...], (((1,), (0,)), ((), ())),
        preferred_element_type=jnp.float32,
    )
    z = z * scale_ref[...]
    n = z.shape[1] // 2
    o_ref[...] = z[:, :n] + z[:, n:]


def _quant_kernel(s_ref, q_ref, scale_ref):
    s = s_ref[...].astype(jnp.float32)
    mh = jnp.max(jnp.abs(s), axis=0, keepdims=True)
    kh = jnp.ceil(jnp.log2(jnp.maximum(mh, 1e-30))) - 8.0
    sch = jnp.exp2(kh)
    qh = (s * jnp.exp2(-kh)).astype(jnp.float8_e4m3fn)
    r = s - qh.astype(jnp.float32) * sch
    ml = jnp.max(jnp.abs(r), axis=0, keepdims=True)
    kl = jnp.ceil(jnp.log2(jnp.maximum(ml, 1e-30))) - 8.0
    scl = jnp.exp2(kl)
    ql = (r * jnp.exp2(-kl)).astype(jnp.float8_e4m3fn)
    q_ref[...] = jnp.concatenate([qh, ql], axis=1)
    scale_ref[...] = jnp.concatenate([sch, scl], axis=1)


def _quantize(s):
    m, n = s.shape
    q, scale = pl.pallas_call(
        _quant_kernel,
        in_specs=[pl.BlockSpec((m, n), lambda: (0, 0))],
        out_specs=[
            pl.BlockSpec((m, 2 * n), lambda: (0, 0)),
            pl.BlockSpec((1, 2 * n), lambda: (0, 0)),
        ],
        out_shape=[
            jax.ShapeDtypeStruct((m, 2 * n), jnp.float8_e4m3fn),
            jax.ShapeDtypeStruct((1, 2 * n), jnp.float32),
        ],
    )(s)
    return q, scale


def _gram_kernel(a_ref, bt_ref, o_ref):
    a = a_ref[...].astype(jnp.bfloat16)
    bt = bt_ref[...].astype(jnp.bfloat16)
    g = jnp.dot(a, bt, preferred_element_type=jnp.float32)
    o_ref[...] = jax.nn.sigmoid(g)


def _support0(x, w1t, act):
    m, _ = x.shape
    n = w1t.shape[1]
    return pl.pallas_call(
        _support0_kernel,
        in_specs=[
            pl.BlockSpec(memory_space=pltpu.SMEM),
            pl.BlockSpec((m, x.shape[1]), lambda: (0, 0)),
            pl.BlockSpec((x.shape[1], n), lambda: (0, 0)),
        ],
        out_specs=pl.BlockSpec((m, n), lambda: (0, 0)),
        out_shape=jax.ShapeDtypeStruct((m, n), jnp.bfloat16),
    )(act, x, w1t)


def _propagate_cast(adj, s, wt, act, *, leaky, out_dtype, compact_dtype):
    m, k = adj.shape
    n = s.shape[1]
    n2 = wt.shape[1]
    bm = _pick_block(m, (200, 80, 16, 8))
    grid = (m // bm,)
    return pl.pallas_call(
        functools.partial(_prop_w_cast_kernel, leaky=leaky),
        grid=grid,
        in_specs=[
            pl.BlockSpec(memory_space=pltpu.SMEM),
            pl.BlockSpec((bm, k), lambda i: (i, 0)),
            pl.BlockSpec((k, n), lambda i: (0, 0)),
            pl.BlockSpec((n, n2), lambda i: (0, 0)),
        ],
        out_specs=[
            pl.BlockSpec((bm, n2), lambda i: (i, 0)),
            pl.BlockSpec((bm, k), lambda i: (i, 0)),
        ],
        out_shape=[
            jax.ShapeDtypeStruct((m, n2), out_dtype),
            jax.ShapeDtypeStruct((m, k), compact_dtype),
        ],
    )(act, adj, s, wt)


def _propagate(adj_c, s, scale, wt, act, *, leaky, out_dtype):
    m, k = adj_c.shape
    n = s.shape[1]
    bm = _pick_block(m, (400, 200, 80, 16, 8))
    grid = (m // bm,)
    if wt is not None:
        nw, n2 = wt.shape
        return pl.pallas_call(
            functools.partial(_prop_w_kernel, leaky=leaky),
            grid=grid,
            in_specs=[
                pl.BlockSpec(memory_space=pltpu.SMEM),
                pl.BlockSpec((1, n), lambda i: (0, 0)),
                pl.BlockSpec((bm, k), lambda i: (i, 0)),
                pl.BlockSpec((k, n), lambda i: (0, 0)),
                pl.BlockSpec((nw, n2), lambda i: (0, 0)),
            ],
            out_specs=pl.BlockSpec((bm, n2), lambda i: (i, 0)),
            out_shape=jax.ShapeDtypeStruct((m, n2), out_dtype),
        )(act, scale, adj_c, s, wt)
    return pl.pallas_call(
        _prop_id_kernel,
        grid=grid,
        in_specs=[
            pl.BlockSpec((1, n), lambda i: (0, 0)),
            pl.BlockSpec((bm, k), lambda i: (i, 0)),
            pl.BlockSpec((k, n), lambda i: (0, 0)),
        ],
        out_specs=pl.BlockSpec((bm, n // 2), lambda i: (i, 0)),
        out_shape=jax.ShapeDtypeStruct((m, n // 2), out_dtype),
    )(scale, adj_c, s)


def _gram_sigmoid(z, zt):
    m = z.shape[0]
    k = z.shape[1]
    bm = _pick_block(m, (400, 200, 80, 16, 8))
    return pl.pallas_call(
        _gram_kernel,
        grid=(m // bm,),
        in_specs=[
            pl.BlockSpec((bm, k), lambda i: (i, 0)),
            pl.BlockSpec((k, m), lambda i: (0, 0)),
        ],
        out_specs=pl.BlockSpec((bm, m), lambda i: (i, 0)),
        out_shape=jax.ShapeDtypeStruct((m, m), jnp.float32),
    )(z, zt)


def kernel(x, adj, active, W1, W2, W3):
    act = jnp.asarray(active, jnp.int32).reshape(1, 1)
    s1 = _support0(x, W1.T, act)
    s2, adj_c = _propagate_cast(
        adj, s1, W2.T, act, leaky=True, out_dtype=jnp.bfloat16,
        compact_dtype=jnp.float8_e4m3fn,
    )
    s2q, sc2 = _quantize(s2)
    s3 = _propagate(adj_c, s2q, sc2, W3.T, act, leaky=False,
                    out_dtype=jnp.bfloat16)
    s3q, sc3 = _quantize(s3)
    z3 = _propagate(adj_c, s3q, sc3, None, None, leaky=False,
                    out_dtype=jnp.float32)
    adj_hat = _gram_sigmoid(z3, z3.T)
    return (z3, adj_hat)


# speedup vs baseline: 1.4168x; 1.0592x over previous
import functools

import jax
import jax.numpy as jnp
from jax.experimental import pallas as pl
from jax.experimental.pallas import tpu as pltpu


def _pick_block(m, candidates):
    for c in candidates:
        if m % c == 0:
            return c
    return m


def _leaky(v):
    return jnp.where(v >= 0.0, v, 0.01 * v)


def _layer1_kernel(act_ref, x_ref, w1_ref, adj_ref, w2_ref,
                   s2_ref, adjc_ref, s1_scr):
    @pl.when(pl.program_id(0) == 0)
    def _():
        s = jnp.dot(x_ref[...], w1_ref[...],
                    preferred_element_type=jnp.float32)
        s = jnp.where(act_ref[0, 0] != 0, _leaky(s), s)
        s1_scr[...] = s.astype(s1_scr.dtype)

    adj_blk = adj_ref[...]
    adjc_ref[...] = adj_blk.astype(adjc_ref.dtype)
    z = jnp.dot(adj_blk.astype(jnp.bfloat16), s1_scr[...],
                preferred_element_type=jnp.float32)
    z = jnp.dot(z, w2_ref[...], preferred_element_type=jnp.float32)
    z = jnp.where(act_ref[0, 0] != 0, _leaky(z), z)
    s2_ref[...] = z.astype(s2_ref.dtype)


def _quant_split(s):
    mh = jnp.max(jnp.abs(s), axis=0, keepdims=True)
    kh = jnp.ceil(jnp.log2(jnp.maximum(mh, 1e-30))) - 8.0
    sch = jnp.exp2(kh)
    qh = (s * jnp.exp2(-kh)).astype(jnp.float8_e4m3fn)
    r = s - qh.astype(jnp.float32) * sch
    ml = jnp.max(jnp.abs(r), axis=0, keepdims=True)
    kl = jnp.ceil(jnp.log2(jnp.maximum(ml, 1e-30))) - 8.0
    scl = jnp.exp2(kl)
    ql = (r * jnp.exp2(-kl)).astype(jnp.float8_e4m3fn)
    return (jnp.concatenate([qh, ql], axis=1),
            jnp.concatenate([sch, scl], axis=1))


def _layer2_kernel(act_ref, s_ref, adj_ref, w_ref, o_ref, q_scr, sc_scr):
    @pl.when(pl.program_id(0) == 0)
    def _():
        q, sc = _quant_split(s_ref[...].astype(jnp.float32))
        q_scr[...] = q
        sc_scr[...] = sc

    z = jax.lax.dot_general(
        adj_ref[...], q_scr[...], (((1,), (0,)), ((), ())),
        preferred_element_type=jnp.float32,
    )
    z = z * sc_scr[...]
    n = z.shape[1] // 2
    z = z[:, :n] + z[:, n:]
    z = jnp.dot(z, w_ref[...], preferred_element_type=jnp.float32)
    o_ref[...] = z.astype(o_ref.dtype)


def _layer3_kernel(s_ref, adj_ref, o_ref, q_scr, sc_scr):
    @pl.when(pl.program_id(0) == 0)
    def _():
        q, sc = _quant_split(s_ref[...].astype(jnp.float32))
        q_scr[...] = q
        sc_scr[...] = sc

    z = jax.lax.dot_general(
        adj_ref[...], q_scr[...], (((1,), (0,)), ((), ())),
        preferred_element_type=jnp.float32,
    )
    z = z * sc_scr[...]
    n = z.shape[1] // 2
    o_ref[...] = z[:, :n] + z[:, n:]


def _gram_kernel(a_ref, bt_ref, o_ref):
    a = a_ref[...].astype(jnp.bfloat16)
    bt = bt_ref[...].astype(jnp.bfloat16)
    g = jnp.dot(a, bt, preferred_element_type=jnp.float32)
    o_ref[...] = jax.nn.sigmoid(g)


def _layer1(adj, x, w1t, w2t, act):
    m, k = adj.shape
    n1 = w1t.shape[1]
    n2 = w2t.shape[1]
    bm = _pick_block(m, (400, 200, 80, 16, 8))
    return pl.pallas_call(
        _layer1_kernel,
        grid=(m // bm,),
        in_specs=[
            pl.BlockSpec(memory_space=pltpu.SMEM),
            pl.BlockSpec((m, x.shape[1]), lambda i: (0, 0)),
            pl.BlockSpec((x.shape[1], n1), lambda i: (0, 0)),
            pl.BlockSpec((bm, k), lambda i: (i, 0)),
            pl.BlockSpec((n1, n2), lambda i: (0, 0)),
        ],
        out_specs=[
            pl.BlockSpec((bm, n2), lambda i: (i, 0)),
            pl.BlockSpec((bm, k), lambda i: (i, 0)),
        ],
        out_shape=[
            jax.ShapeDtypeStruct((m, n2), jnp.bfloat16),
            jax.ShapeDtypeStruct((m, k), jnp.float8_e4m3fn),
        ],
        scratch_shapes=[pltpu.VMEM((m, n1), jnp.bfloat16)],
    )(act, x, w1t, adj, w2t)


def _layer2(adj_c, s, wt, act):
    m, k = adj_c.shape
    n = s.shape[1]
    n2 = wt.shape[1]
    bm = _pick_block(m, (400, 200, 80, 16, 8))
    return pl.pallas_call(
        _layer2_kernel,
        grid=(m // bm,),
        in_specs=[
            pl.BlockSpec(memory_space=pltpu.SMEM),
            pl.BlockSpec((m, n), lambda i: (0, 0)),
            pl.BlockSpec((bm, k), lambda i: (i, 0)),
            pl.BlockSpec((n, n2), lambda i: (0, 0)),
        ],
        out_specs=pl.BlockSpec((bm, n2), lambda i: (i, 0)),
        out_shape=jax.ShapeDtypeStruct((m, n2), jnp.bfloat16),
        scratch_shapes=[
            pltpu.VMEM((m, 2 * n), jnp.float8_e4m3fn),
            pltpu.VMEM((1, 2 * n), jnp.float32),
        ],
    )(act, s, adj_c, wt)


def _layer3(adj_c, s):
    m, k = adj_c.shape
    n = s.shape[1]
    bm = _pick_block(m, (400, 200, 80, 16, 8))
    return pl.pallas_call(
        _layer3_kernel,
        grid=(m // bm,),
        in_specs=[
            pl.BlockSpec((m, n), lambda i: (0, 0)),
            pl.BlockSpec((bm, k), lambda i: (i, 0)),
        ],
        out_specs=pl.BlockSpec((bm, n), lambda i: (i, 0)),
        out_shape=jax.ShapeDtypeStruct((m, n), jnp.float32),
        scratch_shapes=[
            pltpu.VMEM((m, 2 * n), jnp.float8_e4m3fn),
            pltpu.VMEM((1, 2 * n), jnp.float32),
        ],
    )(s, adj_c)


def _gram_sigmoid(z, zt):
    m = z.shape[0]
    k = z.shape[1]
    bm = _pick_block(m, (400, 200, 80, 16, 8))
    return pl.pallas_call(
        _gram_kernel,
        grid=(m // bm,),
        in_specs=[
            pl.BlockSpec((bm, k), lambda i: (i, 0)),
            pl.BlockSpec((k, m), lambda i: (0, 0)),
        ],
        out_specs=pl.BlockSpec((bm, m), lambda i: (i, 0)),
        out_shape=jax.ShapeDtypeStruct((m, m), jnp.float32),
    )(z, zt)


def kernel(x, adj, active, W1, W2, W3):
    act = jnp.asarray(active, jnp.int32).reshape(1, 1)
    s2, adj_c = _layer1(adj, x, W1.T, W2.T, act)
    s3 = _layer2(adj_c, s2, W3.T, act)
    z3 = _layer3(adj_c, s3)
    adj_hat = _gram_sigmoid(z3, z3.T)
    return (z3, adj_hat)
